# baseline (speedup 1.0000x reference)
import functools

import jax
import jax.numpy as jnp
from jax.experimental import pallas as pl
from jax.experimental.pallas import tpu as pltpu


def _gate_kernel(x_ref, w1_ref, b1_ref, w2_ref, b2_ref, o_ref, scale_ref,
                 *, inv_s, dh_half):
    s = pl.program_id(1)
    xt = x_ref[0]

    @pl.when(s == 0)
    def _():
        ssum = jnp.sum(jnp.sum(xt, axis=2), axis=1, keepdims=True)
        smax = jnp.max(jnp.max(xt, axis=2), axis=1, keepdims=True)
        pools = jnp.concatenate([ssum * inv_s, smax], axis=1)
        h = jax.lax.dot_general(w1_ref[...], pools, (((0,), (0,)), ((), ())),
                                preferred_element_type=jnp.float32)
        h = jnp.maximum(h + b1_ref[...], 0.0)
        att2 = jax.lax.dot_general(w2_ref[...], h, (((0,), (0,)), ((), ())),
                                   preferred_element_type=jnp.float32)
        att = att2[:, 0:1] + att2[:, 1:2] + 2.0 * b2_ref[...]
        scale_ref[...] = jnp.reshape(jax.nn.sigmoid(att), scale_ref.shape)

    half = x_ref[0, :, pl.ds(s * dh_half, dh_half), :]
    o_ref[0] = (half * scale_ref[...]).astype(o_ref.dtype)


def kernel(x, w1, b1, w2, b2):
    B, C, D, H, W = x.shape
    S = D * H * W
    DH = D * H
    x4 = x.reshape(B, C, DH, W)
    b1c = b1.astype(jnp.float32).reshape(-1, 1)
    b2c = b2.astype(jnp.float32).reshape(-1, 1)
    Cr = w1.shape[1]
    dh_half = DH // 4

    body = functools.partial(_gate_kernel, inv_s=1.0 / S, dh_half=dh_half)

    itemsize = jnp.dtype(x.dtype).itemsize
    cost = pl.CostEstimate(
        flops=4 * B * C * S,
        transcendentals=B * C,
        bytes_accessed=2 * B * C * S * itemsize)

    out = pl.pallas_call(
        body,
        out_shape=jax.ShapeDtypeStruct((B, C, DH, W), x.dtype),
        grid=(B, 4),
        in_specs=[
            pl.BlockSpec((1, C, DH, W), lambda b, s: (b, 0, 0, 0)),
            pl.BlockSpec((C, Cr), lambda b, s: (0, 0)),
            pl.BlockSpec((Cr, 1), lambda b, s: (0, 0)),
            pl.BlockSpec((Cr, C), lambda b, s: (0, 0)),
            pl.BlockSpec((C, 1), lambda b, s: (0, 0)),
        ],
        out_specs=pl.BlockSpec((1, C, dh_half, W), lambda b, s: (b, 0, s, 0)),
        scratch_shapes=[pltpu.VMEM((C, 1, 1), jnp.float32)],
        compiler_params=pltpu.CompilerParams(
            dimension_semantics=("parallel", "arbitrary")),
        cost_estimate=cost,
    )(x4, w1, b1c, w2, b2c)

    return out.reshape(B, C, D, H, W)


# speedup vs baseline: 5.4629x; 5.4629x over previous
import functools

import jax
import jax.numpy as jnp
from jax.experimental import pallas as pl
from jax.experimental.pallas import tpu as pltpu


def _gate_kernel(x_ref, w1_ref, b1_ref, w2_ref, b2_ref, o_ref, *, inv_s):
    xt = x_ref[0]
    ssum = jnp.sum(xt, axis=0, keepdims=True)
    smax = jnp.max(xt, axis=0, keepdims=True)
    pools = jnp.concatenate([ssum * inv_s, smax], axis=0)
    h = jnp.dot(pools, w1_ref[...],
                preferred_element_type=jnp.float32) + b1_ref[...]
    h = jnp.maximum(h, 0.0)
    att2 = jnp.dot(h, w2_ref[...],
                   preferred_element_type=jnp.float32)
    att = att2[0:1, :] + att2[1:2, :] + 2.0 * b2_ref[...]
    scale = jax.nn.sigmoid(att)
    o_ref[0] = (xt * scale).astype(o_ref.dtype)


def kernel(x, w1, b1, w2, b2):
    B, C, D, H, W = x.shape
    S = D * H * W
    xs = x.transpose(0, 2, 3, 4, 1).reshape(B, S, C)
    b1r = b1.astype(jnp.float32).reshape(1, -1)
    b2r = b2.astype(jnp.float32).reshape(1, -1)
    Cr = w1.shape[1]

    body = functools.partial(_gate_kernel, inv_s=1.0 / S)

    itemsize = jnp.dtype(x.dtype).itemsize
    cost = pl.CostEstimate(
        flops=4 * B * C * S,
        transcendentals=B * C,
        bytes_accessed=2 * B * C * S * itemsize)

    out = pl.pallas_call(
        body,
        out_shape=jax.ShapeDtypeStruct((B, S, C), x.dtype),
        grid=(B,),
        in_specs=[
            pl.BlockSpec((1, S, C), lambda b: (b, 0, 0)),
            pl.BlockSpec((C, Cr), lambda b: (0, 0)),
            pl.BlockSpec((1, Cr), lambda b: (0, 0)),
            pl.BlockSpec((Cr, C), lambda b: (0, 0)),
            pl.BlockSpec((1, C), lambda b: (0, 0)),
        ],
        out_specs=pl.BlockSpec((1, S, C), lambda b: (b, 0, 0)),
        compiler_params=pltpu.CompilerParams(
            dimension_semantics=("parallel",)),
        cost_estimate=cost,
    )(xs, w1, b1r, w2, b2r)

    return out.reshape(B, D, H, W, C).transpose(0, 4, 1, 2, 3)


# speedup vs baseline: 5.6486x; 1.0340x over previous
import functools

import jax
import jax.numpy as jnp
from jax.experimental import pallas as pl
from jax.experimental.pallas import tpu as pltpu


def _gate_kernel(x_ref, w1t_ref, b1_ref, w2_ref, b2_ref, o_ref, *, inv_s):
    xt = x_ref[0]
    ssum = jnp.sum(xt, axis=0, keepdims=True)
    smax = jnp.max(xt, axis=0, keepdims=True)
    pools = jnp.concatenate([ssum * inv_s, smax], axis=0)
    h = jax.lax.dot_general(pools, w1t_ref[...], (((1,), (1,)), ((), ())),
                            preferred_element_type=jnp.float32) + b1_ref[...]
    h = jnp.maximum(h, 0.0)
    att2 = jnp.dot(h, w2_ref[...],
                   preferred_element_type=jnp.float32)
    att = att2[0:1, :] + att2[1:2, :] + 2.0 * b2_ref[...]
    scale = jax.nn.sigmoid(att)
    o_ref[0] = (xt * scale).astype(o_ref.dtype)


def kernel(x, w1, b1, w2, b2):
    B, C, D, H, W = x.shape
    S = D * H * W
    xs = x.transpose(0, 2, 3, 4, 1).reshape(B, S, C)
    w1t = w1.T
    b1r = b1.astype(jnp.float32).reshape(1, -1)
    b2r = b2.astype(jnp.float32).reshape(1, -1)
    Cr = w1.shape[1]

    body = functools.partial(_gate_kernel, inv_s=1.0 / S)

    itemsize = jnp.dtype(x.dtype).itemsize
    cost = pl.CostEstimate(
        flops=4 * B * C * S,
        transcendentals=B * C,
        bytes_accessed=2 * B * C * S * itemsize)

    out = pl.pallas_call(
        body,
        out_shape=jax.ShapeDtypeStruct((B, S, C), x.dtype),
        grid=(B,),
        in_specs=[
            pl.BlockSpec((1, S, C), lambda b: (b, 0, 0)),
            pl.BlockSpec((Cr, C), lambda b: (0, 0)),
            pl.BlockSpec((1, Cr), lambda b: (0, 0)),
            pl.BlockSpec((Cr, C), lambda b: (0, 0)),
            pl.BlockSpec((1, C), lambda b: (0, 0)),
        ],
        out_specs=pl.BlockSpec((1, S, C), lambda b: (b, 0, 0)),
        compiler_params=pltpu.CompilerParams(
            dimension_semantics=("parallel",)),
        cost_estimate=cost,
    )(xs, w1t, b1r, w2, b2r)

    return out.reshape(B, D, H, W, C).transpose(0, 4, 1, 2, 3)
